# Initial kernel scaffold; baseline (speedup 1.0000x reference)
import jax
import jax.numpy as jnp
from jax.experimental import pallas as pl


def kernel(point_cloud, filtration_weights, distance_bias, attn_W, attn_b):
    raise NotImplementedError("write your pallas kernel here")


# speedup vs baseline: 6.0996x; 6.0996x over previous
import jax
import jax.numpy as jnp
from jax import lax
from jax.experimental import pallas as pl

N = 32768
K = 256
R = 256
C = 128
IMIN = -(2 ** 31)
IMAX = 2 ** 31 - 1


def _f2key(f):
    u = lax.bitcast_convert_type(f, jnp.int32)
    lsr = lax.shift_right_logical(u, 31)
    mask = (-lsr) | jnp.int32(IMIN)
    return (u ^ mask) ^ jnp.int32(IMIN)


def _key2f(k):
    u = jnp.where(k >= 0, k, ~(k ^ jnp.int32(IMIN)))
    return lax.bitcast_convert_type(u, jnp.float32)


def _kth_largest_key(keys, k):
    def body(_, lohi):
        lo, hi = lohi
        mid = (lo >> 1) + (hi >> 1) + ((lo | hi) & 1)
        cnt = jnp.sum((keys >= mid).astype(jnp.int32))
        pred = cnt >= k
        return (jnp.where(pred, mid, lo), jnp.where(pred, hi, mid - 1))
    lo, _ = lax.fori_loop(0, 32, body, (jnp.int32(IMIN), jnp.int32(IMAX)))
    return lo


def _kth_smallest_key(keys, k):
    def body(_, lohi):
        lo, hi = lohi
        mid = (lo >> 1) + (hi >> 1) + (lo & hi & 1)
        cnt = jnp.sum((keys <= mid).astype(jnp.int32))
        pred = cnt >= k
        return (jnp.where(pred, lo, mid + 1), jnp.where(pred, mid, hi))
    _, hi = lax.fori_loop(0, 32, body, (jnp.int32(IMIN), jnp.int32(IMAX)))
    return hi


def _tri_upper():
    iu = lax.broadcasted_iota(jnp.int32, (C, C), 0)
    ju = lax.broadcasted_iota(jnp.int32, (C, C), 1)
    return (iu < ju).astype(jnp.float32)


def _tri_lower():
    il = lax.broadcasted_iota(jnp.int32, (R, R), 0)
    jl = lax.broadcasted_iota(jnp.int32, (R, R), 1)
    return (jl < il).astype(jnp.float32)


def _prefix_parts(mask_f):
    within = jnp.dot(mask_f, _tri_upper(), preferred_element_type=jnp.float32)
    rt = jnp.sum(mask_f, axis=1, keepdims=True)
    rtb = jnp.broadcast_to(rt, (R, C))
    row_off = jnp.dot(_tri_lower(), rtb, preferred_element_type=jnp.float32)
    return within, row_off, rtb


def _excl_prefix(mask_f):
    within, row_off, _ = _prefix_parts(mask_f)
    return within + row_off


def _row0_t(a):
    return jnp.transpose(a)[0:1, :]


def _phk(pt_ref, params_ref, out_ref):
    w0 = params_ref[0, 0]
    w1 = params_ref[0, 1]
    b0 = params_ref[0, 2]
    afw = params_ref[0, 3]
    bias = params_ref[0, 4]

    x = pt_ref[0, 0, :].reshape(R, C)
    y = pt_ref[0, 1, :].reshape(R, C)
    s = x * w0 + y * w1 + b0

    keys = _f2key(s)
    T = _kth_largest_key(keys, K)
    gt = keys > T
    n_gt = jnp.sum(gt.astype(jnp.int32))
    eq = keys == T
    eq_rank = _excl_prefix(eq.astype(jnp.float32))
    need = (jnp.int32(K) - n_gt).astype(jnp.float32)
    sel = gt | (eq & (eq_rank < need))

    selF = sel.astype(jnp.float32)
    within, row_off, rtb = _prefix_parts(selF)
    slotf = jnp.where(sel, within + row_off, jnp.float32(-1.0))

    row_offT = _row0_t(row_off)
    rtT = _row0_t(rtb)
    jio = lax.broadcasted_iota(jnp.int32, (K, R), 0).astype(jnp.float32)
    rowOH = ((jio >= row_offT) & (jio < row_offT + rtT)).astype(jnp.float32)
    xrows = jnp.dot(rowOH, x, preferred_element_type=jnp.float32,
                    precision=lax.Precision.HIGHEST)
    yrows = jnp.dot(rowOH, y, preferred_element_type=jnp.float32,
                    precision=lax.Precision.HIGHEST)
    srows = jnp.dot(rowOH, slotf, preferred_element_type=jnp.float32)
    jcol = lax.broadcasted_iota(jnp.int32, (K, 1), 0).astype(jnp.float32)
    laneOH = srows == jcol
    lx = jnp.sum(jnp.where(laneOH, xrows, 0.0), axis=1, keepdims=True)
    ly = jnp.sum(jnp.where(laneOH, yrows, 0.0), axis=1, keepdims=True)
    lxT = _row0_t(jnp.broadcast_to(lx, (K, C)))
    lyT = _row0_t(jnp.broadcast_to(ly, (K, C)))

    dx = lx - lxT
    dy = ly - lyT
    d2 = dx * dx + dy * dy
    safe = jnp.where(d2 > 0, d2, jnp.float32(1.0))
    d = jnp.where(d2 > 0, jnp.sqrt(safe), jnp.float32(0.0))
    sd = d * afw + bias

    n = jnp.float32(K * K)
    s1 = jnp.sum(sd)
    mean = s1 / n
    dev = sd - mean
    std = jnp.sqrt(jnp.sum(dev * dev) / (n - 1.0))
    mn = jnp.min(sd)
    mx = jnp.max(sd)

    kmed = _kth_smallest_key(_f2key(sd), (K * K) // 2)
    thr = _key2f(kmed)
    conn = jnp.sum((sd < thr).astype(jnp.float32)) / n

    rs = jnp.sum(sd, axis=1, keepdims=True)
    mr = jnp.sum(rs) / jnp.float32(K)
    devr = rs - mr
    rstd = jnp.sqrt(jnp.sum(devr * devr) / jnp.float32(K - 1))

    li = lax.broadcasted_iota(jnp.int32, (1, 128), 1)
    vec = jnp.where(li == 0, mean, jnp.float32(0.0))
    vec = jnp.where(li == 1, std, vec)
    vec = jnp.where(li == 2, mn, vec)
    vec = jnp.where(li == 3, mx, vec)
    vec = jnp.where(li == 4, conn, vec)
    vec = jnp.where(li == 5, rstd, vec)
    out_ref[...] = vec.reshape(1, 1, 128)


@jax.jit
def kernel(point_cloud, filtration_weights, distance_bias, attn_W, attn_b):
    B = point_cloud.shape[0]
    pt = jnp.transpose(point_cloud, (0, 2, 1))
    params = jnp.concatenate([
        attn_W[0], attn_b, jnp.abs(filtration_weights)[0], distance_bias])
    params = jnp.pad(params, (0, 128 - params.shape[0])).reshape(1, 128)
    out = pl.pallas_call(
        _phk,
        grid=(B,),
        in_specs=[
            pl.BlockSpec((1, 2, N), lambda i: (i, 0, 0)),
            pl.BlockSpec((1, 128), lambda i: (0, 0)),
        ],
        out_specs=pl.BlockSpec((1, 1, 128), lambda i: (i, 0, 0)),
        out_shape=jax.ShapeDtypeStruct((B, 1, 128), jnp.float32),
    )(pt, params)
    return out[:, 0, :6]


# speedup vs baseline: 6.1034x; 1.0006x over previous
import jax
import jax.numpy as jnp
from jax import lax
from jax.experimental import pallas as pl
from jax.experimental.pallas import tpu as pltpu

N = 32768
K = 256
R = 256
C = 128
IMIN = -(2 ** 31)
IMAX = 2 ** 31 - 1


def _f2key(f):
    u = lax.bitcast_convert_type(f, jnp.int32)
    lsr = lax.shift_right_logical(u, 31)
    mask = (-lsr) | jnp.int32(IMIN)
    return (u ^ mask) ^ jnp.int32(IMIN)


def _key2f(k):
    u = jnp.where(k >= 0, k, ~(k ^ jnp.int32(IMIN)))
    return lax.bitcast_convert_type(u, jnp.float32)


def _kth_largest_key(keys, k):
    def body(_, lohi):
        lo, hi = lohi
        mid = (lo >> 1) + (hi >> 1) + ((lo | hi) & 1)
        cnt = jnp.sum((keys >= mid).astype(jnp.int32))
        pred = cnt >= k
        return (jnp.where(pred, mid, lo), jnp.where(pred, hi, mid - 1))
    lo, _ = lax.fori_loop(0, 32, body, (jnp.int32(IMIN), jnp.int32(IMAX)))
    return lo


def _kth_smallest_key(keys, k):
    def body(_, lohi):
        lo, hi = lohi
        mid = (lo >> 1) + (hi >> 1) + (lo & hi & 1)
        cnt = jnp.sum((keys <= mid).astype(jnp.int32))
        pred = cnt >= k
        return (jnp.where(pred, lo, mid + 1), jnp.where(pred, mid, hi))
    _, hi = lax.fori_loop(0, 32, body, (jnp.int32(IMIN), jnp.int32(IMAX)))
    return hi


def _tri_upper():
    iu = lax.broadcasted_iota(jnp.int32, (C, C), 0)
    ju = lax.broadcasted_iota(jnp.int32, (C, C), 1)
    return (iu < ju).astype(jnp.float32)


def _tri_lower():
    il = lax.broadcasted_iota(jnp.int32, (R, R), 0)
    jl = lax.broadcasted_iota(jnp.int32, (R, R), 1)
    return (jl < il).astype(jnp.float32)


def _prefix_parts(mask_f):
    within = jnp.dot(mask_f, _tri_upper(), preferred_element_type=jnp.float32)
    rt = jnp.sum(mask_f, axis=1, keepdims=True)
    rtb = jnp.broadcast_to(rt, (R, C))
    row_off = jnp.dot(_tri_lower(), rtb, preferred_element_type=jnp.float32)
    return within, row_off, rtb


def _excl_prefix(mask_f):
    within, row_off, _ = _prefix_parts(mask_f)
    return within + row_off


def _row0_t(a):
    return jnp.transpose(a)[0:1, :]


def _phk(pt_ref, params_ref, out_ref):
    w0 = params_ref[0, 0]
    w1 = params_ref[0, 1]
    b0 = params_ref[0, 2]
    afw = params_ref[0, 3]
    bias = params_ref[0, 4]

    x = pt_ref[0, 0, :].reshape(R, C)
    y = pt_ref[0, 1, :].reshape(R, C)
    s = x * w0 + y * w1 + b0

    keys = _f2key(s)
    T = _kth_largest_key(keys, K)
    gt = keys > T
    n_gt = jnp.sum(gt.astype(jnp.int32))
    eq = keys == T
    eq_rank = _excl_prefix(eq.astype(jnp.float32))
    need = (jnp.int32(K) - n_gt).astype(jnp.float32)
    sel = gt | (eq & (eq_rank < need))

    selF = sel.astype(jnp.float32)
    within, row_off, rtb = _prefix_parts(selF)
    slotf = jnp.where(sel, within + row_off, jnp.float32(-1.0))

    row_offT = _row0_t(row_off)
    rtT = _row0_t(rtb)
    jio = lax.broadcasted_iota(jnp.int32, (K, R), 0).astype(jnp.float32)
    rowOH = ((jio >= row_offT) & (jio < row_offT + rtT)).astype(jnp.float32)
    xrows = jnp.dot(rowOH, x, preferred_element_type=jnp.float32,
                    precision=lax.Precision.HIGHEST)
    yrows = jnp.dot(rowOH, y, preferred_element_type=jnp.float32,
                    precision=lax.Precision.HIGHEST)
    srows = jnp.dot(rowOH, slotf, preferred_element_type=jnp.float32)
    jcol = lax.broadcasted_iota(jnp.int32, (K, 1), 0).astype(jnp.float32)
    laneOH = srows == jcol
    lx = jnp.sum(jnp.where(laneOH, xrows, 0.0), axis=1, keepdims=True)
    ly = jnp.sum(jnp.where(laneOH, yrows, 0.0), axis=1, keepdims=True)
    lxT = _row0_t(jnp.broadcast_to(lx, (K, C)))
    lyT = _row0_t(jnp.broadcast_to(ly, (K, C)))

    dx = lx - lxT
    dy = ly - lyT
    d2 = dx * dx + dy * dy
    safe = jnp.where(d2 > 0, d2, jnp.float32(1.0))
    d = jnp.where(d2 > 0, jnp.sqrt(safe), jnp.float32(0.0))
    sd = d * afw + bias

    n = jnp.float32(K * K)
    s1 = jnp.sum(sd)
    mean = s1 / n
    dev = sd - mean
    std = jnp.sqrt(jnp.sum(dev * dev) / (n - 1.0))
    mn = jnp.min(sd)
    mx = jnp.max(sd)

    kmed = _kth_smallest_key(_f2key(sd), (K * K) // 2)
    thr = _key2f(kmed)
    conn = jnp.sum((sd < thr).astype(jnp.float32)) / n

    rs = jnp.sum(sd, axis=1, keepdims=True)
    mr = jnp.sum(rs) / jnp.float32(K)
    devr = rs - mr
    rstd = jnp.sqrt(jnp.sum(devr * devr) / jnp.float32(K - 1))

    li = lax.broadcasted_iota(jnp.int32, (1, 128), 1)
    vec = jnp.where(li == 0, mean, jnp.float32(0.0))
    vec = jnp.where(li == 1, std, vec)
    vec = jnp.where(li == 2, mn, vec)
    vec = jnp.where(li == 3, mx, vec)
    vec = jnp.where(li == 4, conn, vec)
    vec = jnp.where(li == 5, rstd, vec)
    out_ref[...] = vec.reshape(1, 1, 128)


@jax.jit
def kernel(point_cloud, filtration_weights, distance_bias, attn_W, attn_b):
    B = point_cloud.shape[0]
    pt = jnp.transpose(point_cloud, (0, 2, 1))
    params = jnp.concatenate([
        attn_W[0], attn_b, jnp.abs(filtration_weights)[0], distance_bias])
    params = jnp.pad(params, (0, 128 - params.shape[0])).reshape(1, 128)
    out = pl.pallas_call(
        _phk,
        grid=(B,),
        in_specs=[
            pl.BlockSpec((1, 2, N), lambda i: (i, 0, 0)),
            pl.BlockSpec((1, 128), lambda i: (0, 0)),
        ],
        out_specs=pl.BlockSpec((1, 1, 128), lambda i: (i, 0, 0)),
        out_shape=jax.ShapeDtypeStruct((B, 1, 128), jnp.float32),
        compiler_params=pltpu.CompilerParams(
            dimension_semantics=("parallel",)),
    )(pt, params)
    return out[:, 0, :6]


# speedup vs baseline: 6.2972x; 1.0318x over previous
import jax
import jax.numpy as jnp
from jax import lax
from jax.experimental import pallas as pl
from jax.experimental.pallas import tpu as pltpu

N = 32768
K = 256
R = 256
C = 128
IMIN = -(2 ** 31)
IMAX = 2 ** 31 - 1


def _f2key(f):
    u = lax.bitcast_convert_type(f, jnp.int32)
    lsr = lax.shift_right_logical(u, 31)
    mask = (-lsr) | jnp.int32(IMIN)
    return (u ^ mask) ^ jnp.int32(IMIN)


def _key2f(k):
    u = jnp.where(k >= 0, k, ~(k ^ jnp.int32(IMIN)))
    return lax.bitcast_convert_type(u, jnp.float32)


def _kth_largest_key(keys, k):
    def body(_, lohi):
        lo, hi = lohi
        mid = (lo >> 1) + (hi >> 1) + ((lo | hi) & 1)
        cnt = jnp.sum((keys >= mid).astype(jnp.int32))
        pred = cnt >= k
        return (jnp.where(pred, mid, lo), jnp.where(pred, hi, mid - 1))
    lo, _ = lax.fori_loop(0, 32, body, (jnp.int32(IMIN), jnp.int32(IMAX)))
    return lo


def _kth_smallest_key(keys, k):
    def body(_, lohi):
        lo, hi = lohi
        mid = (lo >> 1) + (hi >> 1) + (lo & hi & 1)
        cnt = jnp.sum((keys <= mid).astype(jnp.int32))
        pred = cnt >= k
        return (jnp.where(pred, lo, mid + 1), jnp.where(pred, mid, hi))
    _, hi = lax.fori_loop(0, 32, body, (jnp.int32(IMIN), jnp.int32(IMAX)))
    return hi


def _tri_upper():
    iu = lax.broadcasted_iota(jnp.int32, (C, C), 0)
    ju = lax.broadcasted_iota(jnp.int32, (C, C), 1)
    return (iu < ju).astype(jnp.float32)


def _tri_lower():
    il = lax.broadcasted_iota(jnp.int32, (R, R), 0)
    jl = lax.broadcasted_iota(jnp.int32, (R, R), 1)
    return (jl < il).astype(jnp.float32)


def _prefix_parts(mask_f):
    within = jnp.dot(mask_f, _tri_upper(), preferred_element_type=jnp.float32)
    rt = jnp.sum(mask_f, axis=1, keepdims=True)
    rtb = jnp.broadcast_to(rt, (R, C))
    row_off = jnp.dot(_tri_lower(), rtb, preferred_element_type=jnp.float32)
    return within, row_off, rtb


def _excl_prefix(mask_f):
    within, row_off, _ = _prefix_parts(mask_f)
    return within + row_off


def _row0_t(a):
    return jnp.transpose(a)[0:1, :]


def _phk(pt_ref, params_ref, out_ref):
    w0 = params_ref[0, 0]
    w1 = params_ref[0, 1]
    b0 = params_ref[0, 2]
    afw = params_ref[0, 3]
    bias = params_ref[0, 4]

    x = pt_ref[0, 0, :].reshape(R, C)
    y = pt_ref[0, 1, :].reshape(R, C)
    s = x * w0 + y * w1 + b0

    keys = _f2key(s)
    T = _kth_largest_key(keys, K)
    gt = keys > T
    n_gt = jnp.sum(gt.astype(jnp.int32))
    eq = keys == T
    eq_rank = _excl_prefix(eq.astype(jnp.float32))
    need = (jnp.int32(K) - n_gt).astype(jnp.float32)
    sel = gt | (eq & (eq_rank < need))

    selF = sel.astype(jnp.float32)
    within, row_off, rtb = _prefix_parts(selF)
    slotf = jnp.where(sel, within + row_off, jnp.float32(-1.0))

    row_offT = _row0_t(row_off)
    rtT = _row0_t(rtb)
    jio = lax.broadcasted_iota(jnp.int32, (K, R), 0).astype(jnp.float32)
    rowOH = ((jio >= row_offT) & (jio < row_offT + rtT)).astype(jnp.float32)
    xrows = jnp.dot(rowOH, x, preferred_element_type=jnp.float32)
    yrows = jnp.dot(rowOH, y, preferred_element_type=jnp.float32)
    srows = jnp.dot(rowOH, slotf, preferred_element_type=jnp.float32)
    jcol = lax.broadcasted_iota(jnp.int32, (K, 1), 0).astype(jnp.float32)
    laneOH = srows == jcol
    lx = jnp.sum(jnp.where(laneOH, xrows, 0.0), axis=1, keepdims=True)
    ly = jnp.sum(jnp.where(laneOH, yrows, 0.0), axis=1, keepdims=True)
    lxT = _row0_t(jnp.broadcast_to(lx, (K, C)))
    lyT = _row0_t(jnp.broadcast_to(ly, (K, C)))

    dx = lx - lxT
    dy = ly - lyT
    d2 = dx * dx + dy * dy
    safe = jnp.where(d2 > 0, d2, jnp.float32(1.0))
    d = jnp.where(d2 > 0, jnp.sqrt(safe), jnp.float32(0.0))
    sd = d * afw + bias

    n = jnp.float32(K * K)
    s1 = jnp.sum(sd)
    mean = s1 / n
    dev = sd - mean
    std = jnp.sqrt(jnp.sum(dev * dev) / (n - 1.0))
    mn = jnp.min(sd)
    mx = jnp.max(sd)

    kmed = _kth_smallest_key(_f2key(sd), (K * K) // 2)
    thr = _key2f(kmed)
    conn = jnp.sum((sd < thr).astype(jnp.float32)) / n

    rs = jnp.sum(sd, axis=1, keepdims=True)
    mr = jnp.sum(rs) / jnp.float32(K)
    devr = rs - mr
    rstd = jnp.sqrt(jnp.sum(devr * devr) / jnp.float32(K - 1))

    li = lax.broadcasted_iota(jnp.int32, (1, 128), 1)
    vec = jnp.where(li == 0, mean, jnp.float32(0.0))
    vec = jnp.where(li == 1, std, vec)
    vec = jnp.where(li == 2, mn, vec)
    vec = jnp.where(li == 3, mx, vec)
    vec = jnp.where(li == 4, conn, vec)
    vec = jnp.where(li == 5, rstd, vec)
    out_ref[...] = vec.reshape(1, 1, 128)


@jax.jit
def kernel(point_cloud, filtration_weights, distance_bias, attn_W, attn_b):
    B = point_cloud.shape[0]
    pt = jnp.transpose(point_cloud, (0, 2, 1))
    params = jnp.concatenate([
        attn_W[0], attn_b, jnp.abs(filtration_weights)[0], distance_bias])
    params = jnp.pad(params, (0, 128 - params.shape[0])).reshape(1, 128)
    out = pl.pallas_call(
        _phk,
        grid=(B,),
        in_specs=[
            pl.BlockSpec((1, 2, N), lambda i: (i, 0, 0)),
            pl.BlockSpec((1, 128), lambda i: (0, 0)),
        ],
        out_specs=pl.BlockSpec((1, 1, 128), lambda i: (i, 0, 0)),
        out_shape=jax.ShapeDtypeStruct((B, 1, 128), jnp.float32),
        compiler_params=pltpu.CompilerParams(
            dimension_semantics=("parallel",)),
    )(pt, params)
    return out[:, 0, :6]
